# baseline (speedup 1.0000x reference)
import functools

import numpy as np
import jax
import jax.numpy as jnp
from jax import lax
from jax.experimental import pallas as pl
from jax.experimental.pallas import tpu as pltpu
from jax.experimental.pallas import tpu_sc as plsc

N_OBS = 1_000_000
N_STATES = 16
BATCH = 4096
HIST = 200

NW = 32
ROWS_W = BATCH // NW
CHUNKS = ((0, 96), (96, 104))
CHMAX = 104

NBUCKET = 256
LN2 = float(np.log(2.0))

_i = np.arange(NBUCKET, dtype=np.float64)
_m0 = 1.0 + _i / NBUCKET
_m1 = 1.0 + (_i + 1.0) / NBUCKET
_SLOPE = (np.log(_m1) - np.log(_m0)) / (_m1 - _m0)
_INTERCEPT = np.log(_m0) - _SLOPE * _m0 - 127.0 * np.log(2.0)
_TA = np.asarray(_SLOPE, dtype=np.float32)
_TB = np.asarray(_INTERCEPT, dtype=np.float32)


def _log_rows(rows_ref, slot, n, outb_ref, ta_ref, tb_ref):
    def row_body(r, carry):
        v = rows_ref[slot, r]
        xi = plsc.bitcast(v, jnp.int32)
        eu = jnp.right_shift(xi, 23)
        bk = jnp.bitwise_and(jnp.right_shift(xi, 15), 255)
        mi = jnp.bitwise_or(jnp.bitwise_and(xi, 0x7FFFFF), 0x3F800000)
        m = plsc.bitcast(mi, jnp.float32)
        a = plsc.load_gather(ta_ref, [bk])
        b = plsc.load_gather(tb_ref, [bk])
        outb_ref[r] = eu.astype(jnp.float32) * LN2 + (a * m + b)
        return carry

    lax.fori_loop(0, n, row_body, 0)


def _sc_body(idx_hbm, probs_hbm, ta_hbm, tb_hbm, out_hbm,
             idx_v, rows_v, outb_v, ta_v, tb_v, gsem0, gsem1):
    gsems = (gsem0, gsem1)
    wid = lax.axis_index("s") * 2 + lax.axis_index("c")
    base_row = wid * ROWS_W

    pltpu.sync_copy(ta_hbm, ta_v)
    pltpu.sync_copy(tb_hbm, tb_v)
    pltpu.sync_copy(idx_hbm.at[pl.ds(base_row, ROWS_W)], idx_v)

    def fire(r, half, slot):
        off, size = CHUNKS[half]
        pltpu.async_copy(
            probs_hbm.at[idx_v.at[r, pl.ds(off, size)]],
            rows_v.at[slot, pl.ds(0, size)], gsems[slot])

    def wait(r, half, slot):
        off, size = CHUNKS[half]
        pltpu.make_async_copy(
            probs_hbm.at[idx_v.at[r, pl.ds(off, size)]],
            rows_v.at[slot, pl.ds(0, size)], gsems[slot]).wait()

    fire(0, 0, 0)

    def row_loop(p, carry):
        for s in range(2):
            off, size = CHUNKS[s]
            if s == 0:
                fire(p, 1, 1)
            else:
                @pl.when(p + 1 < ROWS_W)
                def _():
                    fire(p + 1, 0, 0)

            wait(p, s, s)
            _log_rows(rows_v, s, size, outb_v, ta_v, tb_v)
            pltpu.sync_copy(
                outb_v.at[pl.ds(0, size)],
                out_hbm.at[base_row + p, pl.ds(off, size)])
        return carry

    lax.fori_loop(0, ROWS_W, row_loop, 0)


@jax.jit
def kernel(x, probs):
    mesh = plsc.VectorSubcoreMesh(core_axis_name="c", subcore_axis_name="s")
    out = pl.kernel(
        _sc_body,
        out_type=jax.ShapeDtypeStruct((BATCH, HIST, N_STATES), jnp.float32),
        mesh=mesh,
        compiler_params=pltpu.CompilerParams(
            needs_layout_passes=False, use_tc_tiling_on_sc=False),
        scratch_types=[
            pltpu.VMEM((ROWS_W, HIST), jnp.int32),
            pltpu.VMEM((2, CHMAX, N_STATES), jnp.float32),
            pltpu.VMEM((CHMAX, N_STATES), jnp.float32),
            pltpu.VMEM((NBUCKET,), jnp.float32),
            pltpu.VMEM((NBUCKET,), jnp.float32),
            pltpu.SemaphoreType.DMA,
            pltpu.SemaphoreType.DMA,
        ],
    )(x, probs, jnp.asarray(_TA), jnp.asarray(_TB))
    return out


# speedup vs baseline: 1.1275x; 1.1275x over previous
import functools

import numpy as np
import jax
import jax.numpy as jnp
from jax import lax
from jax.experimental import pallas as pl
from jax.experimental.pallas import tpu as pltpu
from jax.experimental.pallas import tpu_sc as plsc

N_OBS = 1_000_000
N_STATES = 16
BATCH = 4096
HIST = 200

NW = 32
BW = BATCH // NW

NBUCKET = 256
LN2 = float(np.log(2.0))

_i = np.arange(NBUCKET, dtype=np.float64)
_m0 = 1.0 + _i / NBUCKET
_m1 = 1.0 + (_i + 1.0) / NBUCKET
_SLOPE = (np.log(_m1) - np.log(_m0)) / (_m1 - _m0)
_INTERCEPT = np.log(_m0) - _SLOPE * _m0 - 127.0 * np.log(2.0)
_TA = np.asarray(_SLOPE, dtype=np.float32)
_TB = np.asarray(_INTERCEPT, dtype=np.float32)

_LANE = np.arange(16, dtype=np.int32)


def _sc_body(xT_hbm, probs_hbm, ta_hbm, tb_hbm, out_hbm,
             idx_v, rows_v, tbuf_v, ta_v, tb_v,
             gsem0, gsem1, osem0, osem1):
    gsems = (gsem0, gsem1)
    osems = (osem0, osem1)
    wid = lax.axis_index("s") * 2 + lax.axis_index("c")
    b0 = wid * BW

    pltpu.sync_copy(ta_hbm, ta_v)
    pltpu.sync_copy(tb_hbm, tb_v)
    pltpu.sync_copy(xT_hbm.at[:, pl.ds(b0, BW)], idx_v)

    def fire(h, slot):
        pltpu.async_copy(probs_hbm.at[idx_v.at[h]], rows_v.at[slot],
                         gsems[slot])

    def wait_gather(h, slot):
        pltpu.make_async_copy(probs_hbm.at[idx_v.at[h]], rows_v.at[slot],
                              gsems[slot]).wait()

    def wait_out(slot):
        pltpu.make_async_copy(tbuf_v.at[slot, pl.ds(0, 8)],
                              out_hbm.at[0, 0, 0], osems[slot]).wait()

    lane = lax.iota(jnp.int32, 16)

    fire(0, 0)

    def pair_body(p, carry):
        for s in range(2):
            h = 2 * p + s
            if s == 0:
                fire(h + 1, 1)
            else:
                @pl.when(p + 1 < HIST // 2)
                def _():
                    fire(h + 1, 0)

            wait_gather(h, s)

            @pl.when(p >= 1)
            def _():
                wait_out(s)
                wait_out(s)

            def row_body(b, carry2):
                v = rows_v[s, b]
                xi = plsc.bitcast(v, jnp.int32)
                eu = jnp.right_shift(xi, 23)
                bk = jnp.bitwise_and(jnp.right_shift(xi, 15), 255)
                mi = jnp.bitwise_or(jnp.bitwise_and(xi, 0x7FFFFF),
                                    0x3F800000)
                m = plsc.bitcast(mi, jnp.float32)
                a = plsc.load_gather(ta_v, [bk])
                bb = plsc.load_gather(tb_v, [bk])
                res = eu.astype(jnp.float32) * LN2 + (a * m + bb)
                plsc.store_scatter(
                    tbuf_v.at[s], [lane, jnp.full((16,), b, jnp.int32)], res)
                return carry2

            lax.fori_loop(0, BW, row_body, 0)

            for ti in range(2):
                pltpu.async_copy(tbuf_v.at[s, pl.ds(ti * 8, 8)],
                                 out_hbm.at[h, ti, wid], osems[s])
        return carry

    lax.fori_loop(0, HIST // 2, pair_body, 0)
    for s in range(2):
        wait_out(s)
        wait_out(s)


@jax.jit
def kernel(x, probs):
    xT = x.T
    mesh = plsc.VectorSubcoreMesh(core_axis_name="c", subcore_axis_name="s")
    out5 = pl.kernel(
        _sc_body,
        out_type=jax.ShapeDtypeStruct((HIST, 2, NW, 8, 128), jnp.float32),
        mesh=mesh,
        compiler_params=pltpu.CompilerParams(
            needs_layout_passes=False, use_tc_tiling_on_sc=False),
        scratch_types=[
            pltpu.VMEM((HIST, BW), jnp.int32),
            pltpu.VMEM((2, BW, N_STATES), jnp.float32),
            pltpu.VMEM((2, N_STATES, 128), jnp.float32),
            pltpu.VMEM((NBUCKET,), jnp.float32),
            pltpu.VMEM((NBUCKET,), jnp.float32),
            pltpu.SemaphoreType.DMA,
            pltpu.SemaphoreType.DMA,
            pltpu.SemaphoreType.DMA,
            pltpu.SemaphoreType.DMA,
        ],
    )(xT, probs, jnp.asarray(_TA), jnp.asarray(_TB))
    return out5.transpose(2, 4, 0, 1, 3).reshape(BATCH, HIST, N_STATES)


# speedup vs baseline: 2.0608x; 1.8277x over previous
import functools

import numpy as np
import jax
import jax.numpy as jnp
from jax import lax
from jax.experimental import pallas as pl
from jax.experimental.pallas import tpu as pltpu
from jax.experimental.pallas import tpu_sc as plsc

N_OBS = 1_000_000
N_STATES = 16
BATCH = 4096
HIST = 200

NW = 32
BW = BATCH // NW

NBUCKET = 4096
LN2 = float(np.log(2.0))
K1 = float(np.log(2.0) / (1 << 23))

_i = np.arange(NBUCKET, dtype=np.float64)
_m0 = 1.0 + _i / NBUCKET
_m1 = 1.0 + (_i + 1.0) / NBUCKET
_c = lambda m: np.log2(m) - (m - 1.0)
_TD = np.asarray(((_c(_m0) + _c(_m1)) * 0.5 - 127.0) * np.log(2.0),
                 dtype=np.float32)


def _sc_body(xT_hbm, probs_hbm, td_hbm, out_hbm,
             idx_v, rows_v, tbuf_v, td_v,
             gsem0, gsem1, osem0, osem1):
    gsems = (gsem0, gsem1)
    osems = (osem0, osem1)
    wid = lax.axis_index("s") * 2 + lax.axis_index("c")
    b0 = wid * BW

    pltpu.sync_copy(td_hbm, td_v)
    pltpu.sync_copy(xT_hbm.at[:, pl.ds(b0, BW)], idx_v)

    def fire(h, slot):
        pltpu.async_copy(probs_hbm.at[idx_v.at[h]], rows_v.at[slot],
                         gsems[slot])

    def wait_gather(h, slot):
        pltpu.make_async_copy(probs_hbm.at[idx_v.at[h]], rows_v.at[slot],
                              gsems[slot]).wait()

    def wait_out(slot):
        pltpu.make_async_copy(tbuf_v.at[slot, pl.ds(0, 8)],
                              out_hbm.at[0, 0, 0], osems[slot]).wait()

    lane = lax.iota(jnp.int32, 16)

    fire(0, 0)

    def pair_body(p, carry):
        for s in range(2):
            h = 2 * p + s
            if s == 0:
                fire(h + 1, 1)
            else:
                @pl.when(p + 1 < HIST // 2)
                def _():
                    fire(h + 1, 0)

            wait_gather(h, s)

            @pl.when(p >= 1)
            def _():
                wait_out(s)
                wait_out(s)

            @functools.partial(plsc.parallel_loop, 0, BW, unroll=8)
            def _(b):
                v = rows_v[s, b]
                xi = plsc.bitcast(v, jnp.int32)
                d = plsc.load_gather(
                    td_v, [jnp.bitwise_and(jnp.right_shift(xi, 11), 4095)])
                res = xi.astype(jnp.float32) * K1 + d
                plsc.store_scatter(
                    tbuf_v.at[s], [lane, jnp.full((16,), b, jnp.int32)], res)

            for ti in range(2):
                pltpu.async_copy(tbuf_v.at[s, pl.ds(ti * 8, 8)],
                                 out_hbm.at[h, ti, wid], osems[s])
        return carry

    lax.fori_loop(0, HIST // 2, pair_body, 0)
    for s in range(2):
        wait_out(s)
        wait_out(s)


@jax.jit
def kernel(x, probs):
    xT = x.T
    mesh = plsc.VectorSubcoreMesh(core_axis_name="c", subcore_axis_name="s")
    out5 = pl.kernel(
        _sc_body,
        out_type=jax.ShapeDtypeStruct((HIST, 2, NW, 8, 128), jnp.float32),
        mesh=mesh,
        compiler_params=pltpu.CompilerParams(
            needs_layout_passes=False, use_tc_tiling_on_sc=False),
        scratch_types=[
            pltpu.VMEM((HIST, BW), jnp.int32),
            pltpu.VMEM((2, BW, N_STATES), jnp.float32),
            pltpu.VMEM((2, N_STATES, 128), jnp.float32),
            pltpu.VMEM((NBUCKET,), jnp.float32),
            pltpu.SemaphoreType.DMA,
            pltpu.SemaphoreType.DMA,
            pltpu.SemaphoreType.DMA,
            pltpu.SemaphoreType.DMA,
        ],
    )(xT, probs, jnp.asarray(_TD))
    return out5.transpose(2, 4, 0, 1, 3).reshape(BATCH, HIST, N_STATES)
